# Initial kernel scaffold; baseline (speedup 1.0000x reference)
import jax
import jax.numpy as jnp
from jax.experimental import pallas as pl


def kernel(x, edge_index, edge_attr, W0, b0, W1, b1, W2, b2, Wfc, bfc):
    raise NotImplementedError("write your pallas kernel here")


# speedup vs baseline: 3.3824x; 3.3824x over previous
import functools

import jax
import jax.numpy as jnp
from jax import lax
from jax.experimental import pallas as pl
from jax.experimental.pallas import tpu as pltpu
from jax.experimental.pallas import tpu_sc as plsc

N = 10000
E = 320000
D = 128
NC = 2
NS = 16
NW = NC * NS
CH = 80
EROWS = E // CH
ROWS_W = EROWS // NW
ROWS_T = EROWS // NS
HROWS = ROWS_T // 2
NPAD = 10240
NODES_T = NPAD // NS
RANGE = NPAD // NC
TRASH = RANGE
ACC_R = RANGE + 8
NODES_C = RANGE // NS


@functools.cache
def _mesh():
    return plsc.VectorSubcoreMesh(core_axis_name="c", subcore_axis_name="s",
                                  num_cores=NC, num_subcores=NS)


_SC_PARAMS = pltpu.CompilerParams(needs_layout_passes=False)


def _deg_body(dst_hbm, w_hbm, out_hbm, dst_v, w_v, buf_v, deg_sp):
    c = lax.axis_index("c")
    s = lax.axis_index("s")
    w = s * NC + c
    zeros = jnp.zeros((16,), jnp.float32)

    pltpu.sync_copy(dst_hbm.at[w], dst_v)
    pltpu.sync_copy(w_hbm.at[w], w_v)

    def zloop(i, _):
        buf_v[pl.ds(i * 16, 16)] = zeros
        return 0
    lax.fori_loop(0, NODES_T // 16, zloop, 0)
    pltpu.sync_copy(buf_v, deg_sp.at[pl.ds(s * NODES_T, NODES_T)])
    plsc.subcore_barrier()

    def dloop(j, _):
        pltpu.sync_copy(w_v.at[j], deg_sp.at[dst_v.at[j]], add=True)
        return 0
    lax.fori_loop(0, ROWS_W, dloop, 0)
    plsc.subcore_barrier()

    pltpu.sync_copy(deg_sp.at[pl.ds(s * NODES_T, NODES_T)],
                    out_hbm.at[c, pl.ds(s * NODES_T, NODES_T)])


def _compute_deg(dst3d, w3d):
    return pl.kernel(
        _deg_body,
        out_type=jax.ShapeDtypeStruct((NC, NPAD), jnp.float32),
        mesh=_mesh(),
        compiler_params=_SC_PARAMS,
        scratch_types=[
            pltpu.VMEM((ROWS_W, CH), jnp.int32),
            pltpu.VMEM((ROWS_W, CH), jnp.float32),
            pltpu.VMEM((NODES_T,), jnp.float32),
            pltpu.VMEM_SHARED((NPAD,), jnp.float32),
        ],
    )(dst3d, w3d)


def _dinv_body(degp_ref, o_ref):
    deg = degp_ref[0] + degp_ref[1]
    o_ref[...] = jnp.where(deg > 0.0, lax.rsqrt(jnp.maximum(deg, 1e-30)), 0.0)


def _compute_dinv(degp):
    return pl.pallas_call(
        _dinv_body,
        in_specs=[pl.BlockSpec((NC, NPAD // D, D), lambda: (0, 0, 0))],
        out_specs=pl.BlockSpec((NPAD // D, D), lambda: (0, 0)),
        out_shape=jax.ShapeDtypeStruct((NPAD // D, D), jnp.float32),
    )(degp.reshape(NC, NPAD // D, D)).reshape(NPAD)


def _norm_body(src_hbm, dst_hbm, w_hbm, dinv_hbm, norm_hbm,
               src_v, dst_v, w_v, dinv_full):
    c = lax.axis_index("c")
    s = lax.axis_index("s")
    w = s * NC + c

    pltpu.sync_copy(src_hbm.at[w], src_v)
    pltpu.sync_copy(dst_hbm.at[w], dst_v)
    pltpu.sync_copy(w_hbm.at[w], w_v)
    pltpu.sync_copy(dinv_hbm, dinv_full)

    def nloop(j, _):
        def gloop(g, _):
            sl = pl.ds(g * 16, 16)
            si = src_v[j, sl]
            di = dst_v[j, sl]
            wv = w_v[j, sl]
            nv = (plsc.load_gather(dinv_full, [si]) * wv
                  * plsc.load_gather(dinv_full, [di]))
            w_v[j, sl] = nv
            return 0
        lax.fori_loop(0, CH // 16, gloop, 0)
        return 0
    lax.fori_loop(0, ROWS_W, nloop, 0)
    pltpu.sync_copy(w_v, norm_hbm.at[w])


def _compute_norm(src3d, dst3d, w3d, dinv):
    return pl.kernel(
        _norm_body,
        out_type=jax.ShapeDtypeStruct((NW, ROWS_W, CH), jnp.float32),
        mesh=_mesh(),
        compiler_params=_SC_PARAMS,
        scratch_types=[
            pltpu.VMEM((ROWS_W, CH), jnp.int32),
            pltpu.VMEM((ROWS_W, CH), jnp.int32),
            pltpu.VMEM((ROWS_W, CH), jnp.float32),
            pltpu.VMEM((NPAD,), jnp.float32),
        ],
    )(src3d, dst3d, w3d, dinv)


def _prop_body(h_hbm, src_hbm, dstc_hbm, nrm_hbm, out_hbm,
               src_v, dst_v, nrm_v, rows_v, zbuf, acc_sp, sem):
    c = lax.axis_index("c")
    s = lax.axis_index("s")
    widx = c * NS + s
    zeros = jnp.zeros((16,), jnp.float32)

    def zloop(i, _):
        def zg(g, _):
            zbuf[i, pl.ds(g * 16, 16)] = zeros
            return 0
        lax.fori_loop(0, D // 16, zg, 0)
        return 0
    lax.fori_loop(0, CH, zloop, 0)
    for k in range(NODES_C // CH):
        pltpu.sync_copy(zbuf, acc_sp.at[pl.ds(s * NODES_C + k * CH, CH)])
    @pl.when(s == 0)
    def _():
        pltpu.sync_copy(zbuf.at[pl.ds(0, ACC_R - RANGE)],
                        acc_sp.at[pl.ds(RANGE, ACC_R - RANGE)])
    plsc.subcore_barrier()

    for half in range(2):
        pltpu.sync_copy(src_hbm.at[s, half], src_v)
        pltpu.sync_copy(dstc_hbm.at[widx, half], dst_v)
        pltpu.sync_copy(nrm_hbm.at[s, half], nrm_v)

        def eloop(j, _):
            pltpu.async_copy(h_hbm.at[src_v.at[j]], rows_v, sem).wait()

            def iloop(i, _):
                nb = plsc.load_gather(
                    nrm_v, [jnp.full((16,), j, jnp.int32),
                            jnp.full((16,), i, jnp.int32)])
                for g in range(D // 16):
                    sl = pl.ds(g * 16, 16)
                    rows_v[i, sl] = rows_v[i, sl] * nb
                return 0
            lax.fori_loop(0, CH, iloop, 0)
            pltpu.sync_copy(rows_v, acc_sp.at[dst_v.at[j]], add=True)
            return 0
        lax.fori_loop(0, HROWS, eloop, 0)
    plsc.subcore_barrier()

    pltpu.sync_copy(acc_sp.at[pl.ds(s * NODES_C, NODES_C)],
                    out_hbm.at[c, pl.ds(s * NODES_C, NODES_C)])


def _propagate(h, src16, dstc, nrm16):
    return pl.kernel(
        _prop_body,
        out_type=jax.ShapeDtypeStruct((NC, RANGE, D), jnp.float32),
        mesh=_mesh(),
        compiler_params=_SC_PARAMS,
        scratch_types=[
            pltpu.VMEM((HROWS, CH), jnp.int32),
            pltpu.VMEM((HROWS, CH), jnp.int32),
            pltpu.VMEM((HROWS, CH), jnp.float32),
            pltpu.VMEM((CH, D), jnp.float32),
            pltpu.VMEM((CH, D), jnp.float32),
            pltpu.VMEM_SHARED((ACC_R, D), jnp.float32),
            pltpu.SemaphoreType.DMA,
        ],
    )(h, src16, dstc, nrm16)


ROWS_B = 1280


def _tag_mm_body(h0, h1, h2, h3, w_ref, b_ref, o_ref, *, relu):
    acc = jnp.dot(h0[...], w_ref[0], preferred_element_type=jnp.float32)
    acc += jnp.dot(h1[...], w_ref[1], preferred_element_type=jnp.float32)
    acc += jnp.dot(h2[...], w_ref[2], preferred_element_type=jnp.float32)
    acc += jnp.dot(h3[...], w_ref[3], preferred_element_type=jnp.float32)
    acc += b_ref[...]
    if relu:
        acc = jnp.maximum(acc, 0.0)
    o_ref[...] = acc


def _tag_mm(h0, h1, h2, h3, W, b, relu):
    blk = lambda i: (i, 0)
    return pl.pallas_call(
        functools.partial(_tag_mm_body, relu=relu),
        grid=(NPAD // ROWS_B,),
        in_specs=[
            pl.BlockSpec((ROWS_B, D), blk),
            pl.BlockSpec((ROWS_B, D), blk),
            pl.BlockSpec((ROWS_B, D), blk),
            pl.BlockSpec((ROWS_B, D), blk),
            pl.BlockSpec((4, D, D), lambda i: (0, 0, 0)),
            pl.BlockSpec((1, D), lambda i: (0, 0)),
        ],
        out_specs=pl.BlockSpec((ROWS_B, D), blk),
        out_shape=jax.ShapeDtypeStruct((NPAD, D), jnp.float32),
    )(h0, h1, h2, h3, W, b.reshape(1, D))


def _tag_mm_final_body(h0, h1, h2, h3, w_ref, b_ref, wfc_ref, bfc_ref, o_ref):
    acc = jnp.dot(h0[...], w_ref[0], preferred_element_type=jnp.float32)
    acc += jnp.dot(h1[...], w_ref[1], preferred_element_type=jnp.float32)
    acc += jnp.dot(h2[...], w_ref[2], preferred_element_type=jnp.float32)
    acc += jnp.dot(h3[...], w_ref[3], preferred_element_type=jnp.float32)
    acc += b_ref[...]
    acc = jnp.dot(acc, wfc_ref[...], preferred_element_type=jnp.float32)
    o_ref[...] = acc + bfc_ref[...]


def _tag_mm_final(h0, h1, h2, h3, W, b, Wfc, bfc):
    blk = lambda i: (i, 0)
    return pl.pallas_call(
        _tag_mm_final_body,
        grid=(NPAD // ROWS_B,),
        in_specs=[
            pl.BlockSpec((ROWS_B, D), blk),
            pl.BlockSpec((ROWS_B, D), blk),
            pl.BlockSpec((ROWS_B, D), blk),
            pl.BlockSpec((ROWS_B, D), blk),
            pl.BlockSpec((4, D, D), lambda i: (0, 0, 0)),
            pl.BlockSpec((1, D), lambda i: (0, 0)),
            pl.BlockSpec((D, D), lambda i: (0, 0)),
            pl.BlockSpec((1, D), lambda i: (0, 0)),
        ],
        out_specs=pl.BlockSpec((ROWS_B, D), blk),
        out_shape=jax.ShapeDtypeStruct((NPAD, D), jnp.float32),
    )(h0, h1, h2, h3, W, b.reshape(1, D), Wfc, bfc.reshape(1, D))


def kernel(x, edge_index, edge_attr, W0, b0, W1, b1, W2, b2, Wfc, bfc):
    src = edge_index[0]
    dst = edge_index[1]
    w = edge_attr.reshape(E)

    src32 = src.reshape(NW, ROWS_W, CH)
    dst32 = dst.reshape(NW, ROWS_W, CH)
    w32 = w.reshape(NW, ROWS_W, CH)

    degp = _compute_deg(dst32, w32)
    dinv = _compute_dinv(degp)
    nrm = _compute_norm(src32, dst32, w32, dinv)

    src16 = src.reshape(NS, 2, HROWS, CH)
    nrm16 = nrm.reshape(NS, 2, HROWS, CH)
    parts = []
    for c in range(NC):
        lo = c * RANGE
        inr = (dst >= lo) & (dst < lo + RANGE)
        parts.append(jnp.where(inr, dst - lo, TRASH)
                     .reshape(1, NS, 2, HROWS, CH))
    dstc = jnp.concatenate(parts, axis=0).reshape(NW, 2, HROWS, CH)

    xp = jnp.concatenate([x, jnp.zeros((NPAD - N, D), jnp.float32)], axis=0)
    h = xp
    for li, (W, b) in enumerate(((W0, b0), (W1, b1), (W2, b2))):
        p1 = _propagate(h, src16, dstc, nrm16).reshape(NPAD, D)
        p2 = _propagate(p1, src16, dstc, nrm16).reshape(NPAD, D)
        p3 = _propagate(p2, src16, dstc, nrm16).reshape(NPAD, D)
        if li == 2:
            out = _tag_mm_final(h, p1, p2, p3, W, b, Wfc, bfc)
        else:
            h = _tag_mm(h, p1, p2, p3, W, b, relu=True)
    return out[:N]


# speedup vs baseline: 5.9245x; 1.7516x over previous
import functools

import jax
import jax.numpy as jnp
from jax import lax
from jax.experimental import pallas as pl
from jax.experimental.pallas import tpu as pltpu
from jax.experimental.pallas import tpu_sc as plsc

N = 10000
E = 320000
D = 128
NC = 2
NS = 16
NW = NC * NS
CH = 80
EROWS = E // CH
ROWS_W = EROWS // NW
ROWS_T = EROWS // NS
HROWS = ROWS_T // 2
NPAD = 10240
NODES_T = NPAD // NS
RANGE = NPAD // NC
TRASH = RANGE
ACC_R = RANGE + 8
NODES_C = RANGE // NS


@functools.cache
def _mesh():
    return plsc.VectorSubcoreMesh(core_axis_name="c", subcore_axis_name="s",
                                  num_cores=NC, num_subcores=NS)


_SC_PARAMS = pltpu.CompilerParams(needs_layout_passes=False)


def _deg_body(dst_hbm, w_hbm, out_hbm, dst_v, w_v, buf_v, deg_sp):
    c = lax.axis_index("c")
    s = lax.axis_index("s")
    w = s * NC + c
    zeros = jnp.zeros((16,), jnp.float32)

    pltpu.sync_copy(dst_hbm.at[w], dst_v)
    pltpu.sync_copy(w_hbm.at[w], w_v)

    def zloop(i, _):
        buf_v[pl.ds(i * 16, 16)] = zeros
        return 0
    lax.fori_loop(0, NODES_T // 16, zloop, 0)
    pltpu.sync_copy(buf_v, deg_sp.at[pl.ds(s * NODES_T, NODES_T)])
    plsc.subcore_barrier()

    def dloop(j, _):
        pltpu.sync_copy(w_v.at[j], deg_sp.at[dst_v.at[j]], add=True)
        return 0
    lax.fori_loop(0, ROWS_W, dloop, 0)
    plsc.subcore_barrier()

    pltpu.sync_copy(deg_sp.at[pl.ds(s * NODES_T, NODES_T)],
                    out_hbm.at[c, pl.ds(s * NODES_T, NODES_T)])


def _compute_deg(dst3d, w3d):
    return pl.kernel(
        _deg_body,
        out_type=jax.ShapeDtypeStruct((NC, NPAD), jnp.float32),
        mesh=_mesh(),
        compiler_params=_SC_PARAMS,
        scratch_types=[
            pltpu.VMEM((ROWS_W, CH), jnp.int32),
            pltpu.VMEM((ROWS_W, CH), jnp.float32),
            pltpu.VMEM((NODES_T,), jnp.float32),
            pltpu.VMEM_SHARED((NPAD,), jnp.float32),
        ],
    )(dst3d, w3d)


def _dinv_body(degp_ref, o_ref):
    deg = degp_ref[0] + degp_ref[1]
    o_ref[...] = jnp.where(deg > 0.0, lax.rsqrt(jnp.maximum(deg, 1e-30)), 0.0)


def _compute_dinv(degp):
    return pl.pallas_call(
        _dinv_body,
        in_specs=[pl.BlockSpec((NC, NPAD // D, D), lambda: (0, 0, 0))],
        out_specs=pl.BlockSpec((NPAD // D, D), lambda: (0, 0)),
        out_shape=jax.ShapeDtypeStruct((NPAD // D, D), jnp.float32),
    )(degp.reshape(NC, NPAD // D, D)).reshape(NPAD)


def _norm_body(src_hbm, dst_hbm, w_hbm, dinv_hbm, norm_hbm,
               src_v, dst_v, w_v, dinv_full):
    c = lax.axis_index("c")
    s = lax.axis_index("s")
    w = s * NC + c

    pltpu.sync_copy(src_hbm.at[w], src_v)
    pltpu.sync_copy(dst_hbm.at[w], dst_v)
    pltpu.sync_copy(w_hbm.at[w], w_v)
    pltpu.sync_copy(dinv_hbm, dinv_full)

    def nloop(j, _):
        def gloop(g, _):
            sl = pl.ds(g * 16, 16)
            si = src_v[j, sl]
            di = dst_v[j, sl]
            wv = w_v[j, sl]
            nv = (plsc.load_gather(dinv_full, [si]) * wv
                  * plsc.load_gather(dinv_full, [di]))
            w_v[j, sl] = nv
            return 0
        lax.fori_loop(0, CH // 16, gloop, 0)
        return 0
    lax.fori_loop(0, ROWS_W, nloop, 0)
    pltpu.sync_copy(w_v, norm_hbm.at[w])


def _compute_norm(src3d, dst3d, w3d, dinv):
    return pl.kernel(
        _norm_body,
        out_type=jax.ShapeDtypeStruct((NW, ROWS_W, CH), jnp.float32),
        mesh=_mesh(),
        compiler_params=_SC_PARAMS,
        scratch_types=[
            pltpu.VMEM((ROWS_W, CH), jnp.int32),
            pltpu.VMEM((ROWS_W, CH), jnp.int32),
            pltpu.VMEM((ROWS_W, CH), jnp.float32),
            pltpu.VMEM((NPAD,), jnp.float32),
        ],
    )(src3d, dst3d, w3d, dinv)


def _prop_body(h_hbm, src_hbm, dstc_hbm, nrm_hbm, out_hbm,
               src_v, dst_v, nrm_v, rows_a, rows_b, acc_sp, sem):
    c = lax.axis_index("c")
    s = lax.axis_index("s")
    widx = c * NS + s
    zeros = jnp.zeros((16,), jnp.float32)

    def zloop(i, _):
        def zg(g, _):
            rows_a[i, pl.ds(g * 16, 16)] = zeros
            return 0
        lax.fori_loop(0, D // 16, zg, 0)
        return 0
    lax.fori_loop(0, CH, zloop, 0)
    for k in range(NODES_C // CH):
        pltpu.sync_copy(rows_a, acc_sp.at[pl.ds(s * NODES_C + k * CH, CH)])
    @pl.when(s == 0)
    def _():
        pltpu.sync_copy(rows_a.at[pl.ds(0, ACC_R - RANGE)],
                        acc_sp.at[pl.ds(RANGE, ACC_R - RANGE)])
    plsc.subcore_barrier()

    def scale_scatter(buf, j):
        jv = jnp.full((16,), j, jnp.int32)

        def iloop(i, _):
            nb = plsc.load_gather(nrm_v, [jv, jnp.full((16,), i, jnp.int32)])
            for g in range(D // 16):
                sl = pl.ds(g * 16, 16)
                buf[i, sl] = buf[i, sl] * nb
            return 0
        lax.fori_loop(0, CH, iloop, 0)
        pltpu.sync_copy(buf, acc_sp.at[dst_v.at[j]], add=True)

    for half in range(2):
        pltpu.sync_copy(src_hbm.at[s, half], src_v)
        pltpu.sync_copy(dstc_hbm.at[widx, half], dst_v)
        pltpu.sync_copy(nrm_hbm.at[s, half], nrm_v)

        pltpu.async_copy(h_hbm.at[src_v.at[0]], rows_a, sem)

        def ploop(t, _):
            j0 = 2 * t
            pltpu.make_async_copy(h_hbm.at[src_v.at[0]], rows_a, sem).wait()
            pltpu.async_copy(h_hbm.at[src_v.at[j0 + 1]], rows_b, sem)
            scale_scatter(rows_a, j0)
            pltpu.make_async_copy(h_hbm.at[src_v.at[0]], rows_b, sem).wait()
            pltpu.async_copy(h_hbm.at[src_v.at[j0 + 2]], rows_a, sem)
            scale_scatter(rows_b, j0 + 1)
            return 0
        lax.fori_loop(0, (HROWS - 1) // 2, ploop, 0)
        pltpu.make_async_copy(h_hbm.at[src_v.at[0]], rows_a, sem).wait()
        scale_scatter(rows_a, HROWS - 1)
    plsc.subcore_barrier()

    pltpu.sync_copy(acc_sp.at[pl.ds(s * NODES_C, NODES_C)],
                    out_hbm.at[c, pl.ds(s * NODES_C, NODES_C)])


def _propagate(h, src16, dstc, nrm16):
    return pl.kernel(
        _prop_body,
        out_type=jax.ShapeDtypeStruct((NC, RANGE, D), jnp.float32),
        mesh=_mesh(),
        compiler_params=_SC_PARAMS,
        scratch_types=[
            pltpu.VMEM((HROWS, CH), jnp.int32),
            pltpu.VMEM((HROWS, CH), jnp.int32),
            pltpu.VMEM((HROWS, CH), jnp.float32),
            pltpu.VMEM((CH, D), jnp.float32),
            pltpu.VMEM((CH, D), jnp.float32),
            pltpu.VMEM_SHARED((ACC_R, D), jnp.float32),
            pltpu.SemaphoreType.DMA,
        ],
    )(h, src16, dstc, nrm16)


ROWS_B = 1280


def _tag_mm_body(h0, h1, h2, h3, w_ref, b_ref, o_ref, *, relu):
    acc = jnp.dot(h0[...], w_ref[0], preferred_element_type=jnp.float32)
    acc += jnp.dot(h1[...], w_ref[1], preferred_element_type=jnp.float32)
    acc += jnp.dot(h2[...], w_ref[2], preferred_element_type=jnp.float32)
    acc += jnp.dot(h3[...], w_ref[3], preferred_element_type=jnp.float32)
    acc += b_ref[...]
    if relu:
        acc = jnp.maximum(acc, 0.0)
    o_ref[...] = acc


def _tag_mm(h0, h1, h2, h3, W, b, relu):
    blk = lambda i: (i, 0)
    return pl.pallas_call(
        functools.partial(_tag_mm_body, relu=relu),
        grid=(NPAD // ROWS_B,),
        in_specs=[
            pl.BlockSpec((ROWS_B, D), blk),
            pl.BlockSpec((ROWS_B, D), blk),
            pl.BlockSpec((ROWS_B, D), blk),
            pl.BlockSpec((ROWS_B, D), blk),
            pl.BlockSpec((4, D, D), lambda i: (0, 0, 0)),
            pl.BlockSpec((1, D), lambda i: (0, 0)),
        ],
        out_specs=pl.BlockSpec((ROWS_B, D), blk),
        out_shape=jax.ShapeDtypeStruct((NPAD, D), jnp.float32),
    )(h0, h1, h2, h3, W, b.reshape(1, D))


def _tag_mm_final_body(h0, h1, h2, h3, w_ref, b_ref, wfc_ref, bfc_ref, o_ref):
    acc = jnp.dot(h0[...], w_ref[0], preferred_element_type=jnp.float32)
    acc += jnp.dot(h1[...], w_ref[1], preferred_element_type=jnp.float32)
    acc += jnp.dot(h2[...], w_ref[2], preferred_element_type=jnp.float32)
    acc += jnp.dot(h3[...], w_ref[3], preferred_element_type=jnp.float32)
    acc += b_ref[...]
    acc = jnp.dot(acc, wfc_ref[...], preferred_element_type=jnp.float32)
    o_ref[...] = acc + bfc_ref[...]


def _tag_mm_final(h0, h1, h2, h3, W, b, Wfc, bfc):
    blk = lambda i: (i, 0)
    return pl.pallas_call(
        _tag_mm_final_body,
        grid=(NPAD // ROWS_B,),
        in_specs=[
            pl.BlockSpec((ROWS_B, D), blk),
            pl.BlockSpec((ROWS_B, D), blk),
            pl.BlockSpec((ROWS_B, D), blk),
            pl.BlockSpec((ROWS_B, D), blk),
            pl.BlockSpec((4, D, D), lambda i: (0, 0, 0)),
            pl.BlockSpec((1, D), lambda i: (0, 0)),
            pl.BlockSpec((D, D), lambda i: (0, 0)),
            pl.BlockSpec((1, D), lambda i: (0, 0)),
        ],
        out_specs=pl.BlockSpec((ROWS_B, D), blk),
        out_shape=jax.ShapeDtypeStruct((NPAD, D), jnp.float32),
    )(h0, h1, h2, h3, W, b.reshape(1, D), Wfc, bfc.reshape(1, D))


def kernel(x, edge_index, edge_attr, W0, b0, W1, b1, W2, b2, Wfc, bfc):
    src = edge_index[0]
    dst = edge_index[1]
    w = edge_attr.reshape(E)

    src32 = src.reshape(NW, ROWS_W, CH)
    dst32 = dst.reshape(NW, ROWS_W, CH)
    w32 = w.reshape(NW, ROWS_W, CH)

    degp = _compute_deg(dst32, w32)
    dinv = _compute_dinv(degp)
    nrm = _compute_norm(src32, dst32, w32, dinv)

    src16 = src.reshape(NS, 2, HROWS, CH)
    nrm16 = nrm.reshape(NS, 2, HROWS, CH)
    parts = []
    for c in range(NC):
        lo = c * RANGE
        inr = (dst >= lo) & (dst < lo + RANGE)
        parts.append(jnp.where(inr, dst - lo, TRASH)
                     .reshape(1, NS, 2, HROWS, CH))
    dstc = jnp.concatenate(parts, axis=0).reshape(NW, 2, HROWS, CH)

    xp = jnp.concatenate([x, jnp.zeros((NPAD - N, D), jnp.float32)], axis=0)
    h = xp
    for li, (W, b) in enumerate(((W0, b0), (W1, b1), (W2, b2))):
        p1 = _propagate(h, src16, dstc, nrm16).reshape(NPAD, D)
        p2 = _propagate(p1, src16, dstc, nrm16).reshape(NPAD, D)
        p3 = _propagate(p2, src16, dstc, nrm16).reshape(NPAD, D)
        if li == 2:
            out = _tag_mm_final(h, p1, p2, p3, W, b, Wfc, bfc)
        else:
            h = _tag_mm(h, p1, p2, p3, W, b, relu=True)
    return out[:N]
